# baseline (speedup 1.0000x reference)
import functools
import jax
import jax.numpy as jnp
import numpy as np
from jax.experimental import pallas as pl

_BLOCK = 8
_MIN_E = 0.2
_MAX_E = 0.6
_STRENGTH = 0.5


def _dct_matrix_np(n):
    k = np.arange(n)[:, None]
    m = np.arange(n)[None, :]
    D = np.sqrt(2.0 / n) * np.cos(np.pi * (2 * m + 1) * k / (2 * n))
    D[0, :] = D[0, :] / np.sqrt(2.0)
    return D.astype(np.float32)


def _blockify(x, bs):
    B, C, H, W = x.shape
    return x.reshape(B, C, H // bs, bs, W // bs, bs).transpose(0, 1, 2, 4, 3, 5)


def _unblockify(blocks, H, W):
    B, C, nh, nw, bs, _ = blocks.shape
    return blocks.transpose(0, 1, 2, 4, 3, 5).reshape(B, C, H, W)


def _chaotic_mask_np(bs):
    x = 0.37
    vals = []
    for _ in range(bs * bs):
        x = 3.99 * x * (1.0 - x)
        vals.append(x)
    return (np.array(vals) > 0.5).astype(np.float32).reshape(bs, bs)


def _dct2_kernel(a_ref, p_ref, pt_ref, o_ref):
    A = a_ref[0]
    t = jnp.dot(p_ref[...], A, preferred_element_type=jnp.float32)
    o_ref[0] = jnp.dot(t, pt_ref[...], preferred_element_type=jnp.float32)


def _dct2_pallas(imgs, P, PT):
    N, H, W = imgs.shape
    return pl.pallas_call(
        _dct2_kernel,
        grid=(N,),
        in_specs=[
            pl.BlockSpec((1, H, W), lambda i: (i, 0, 0)),
            pl.BlockSpec((H, H), lambda i: (0, 0)),
            pl.BlockSpec((H, H), lambda i: (0, 0)),
        ],
        out_specs=pl.BlockSpec((1, H, W), lambda i: (i, 0, 0)),
        out_shape=jax.ShapeDtypeStruct((N, H, W), jnp.float32),
    )(imgs, P, PT)


def kernel(cover, secret_bits):
    B, C, H, W = cover.shape
    bs = _BLOCK
    Dnp = _dct_matrix_np(bs)
    nb = H // bs
    P = jnp.asarray(np.kron(np.eye(nb, dtype=np.float32), Dnp))
    PT = jnp.asarray(np.kron(np.eye(nb, dtype=np.float32), Dnp.T.copy()))

    imgs = cover.reshape(B * C, H, W)
    dct_img = _dct2_pallas(imgs, P, PT)
    dct_blocks = (
        dct_img.reshape(B, C, nb, bs, nb, bs).transpose(0, 1, 2, 4, 3, 5)
    )

    D = jnp.asarray(Dnp)
    a = jnp.abs(dct_blocks)
    bmax = jnp.max(a, axis=(-2, -1), keepdims=True)
    e = a / (bmax + 1e-8)
    mask = ((e >= _MIN_E) & (e <= _MAX_E)).astype(jnp.float32)
    mask = mask.at[..., 0, 0].set(0.0)
    mask = mask * jnp.asarray(_chaotic_mask_np(bs))
    tv = jnp.var(_blockify(cover, bs), axis=(-2, -1))
    vn = (tv - tv.min()) / (tv.max() - tv.min() + 1e-8)
    thr = jnp.quantile(vn.reshape(-1), 0.3)
    tmask = (vn > thr).astype(jnp.float32)[..., None, None]
    mask = mask * tmask
    num_bits = secret_bits.shape[1]
    flat_mask = mask.reshape(-1) > 0
    total = flat_mask.shape[0]
    per_batch = total // B
    rank = jnp.cumsum(flat_mask.astype(jnp.int32)) - 1
    selected = flat_mask & (rank < num_bits)
    b_idx_all = jnp.arange(total) // per_batch
    rank_safe = jnp.clip(rank, 0, num_bits - 1)
    bits_all = secret_bits[b_idx_all, rank_safe].astype(jnp.float32)
    flat = dct_blocks.reshape(-1)
    c = flat
    rounded = jnp.round(c)
    lsb = jnp.mod(jnp.abs(rounded), 2.0)
    need = jnp.not_equal(lsb, bits_all)
    delta = jnp.where(selected & need, _STRENGTH * (2.0 * bits_all - 1.0) * jnp.where(c >= 0, 1.0, -1.0), 0.0)
    flat = flat + delta
    modified_blocks = flat.reshape(mask.shape)
    emap = selected.astype(jnp.float32).reshape(mask.shape)
    modified_dct = _unblockify(modified_blocks, H, W)
    mimgs = modified_dct.reshape(B * C, H, W)
    stego_img = _dct2_pallas(mimgs, PT, P)
    stego = stego_img.reshape(B, C, H, W)
    return stego, emap


# speedup vs baseline: 1.0130x; 1.0130x over previous
import functools
import jax
import jax.numpy as jnp
import numpy as np
from jax.experimental import pallas as pl

_BLOCK = 8
_MIN_E = 0.2
_MAX_E = 0.6
_STRENGTH = 0.5


def _dct_matrix_np(n):
    k = np.arange(n)[:, None]
    m = np.arange(n)[None, :]
    D = np.sqrt(2.0 / n) * np.cos(np.pi * (2 * m + 1) * k / (2 * n))
    D[0, :] = D[0, :] / np.sqrt(2.0)
    return D.astype(np.float32)


def _blockify(x, bs):
    B, C, H, W = x.shape
    return x.reshape(B, C, H // bs, bs, W // bs, bs).transpose(0, 1, 2, 4, 3, 5)


def _unblockify(blocks, H, W):
    B, C, nh, nw, bs, _ = blocks.shape
    return blocks.transpose(0, 1, 2, 4, 3, 5).reshape(B, C, H, W)


def _chaotic_mask_np(bs):
    x = 0.37
    vals = []
    for _ in range(bs * bs):
        x = 3.99 * x * (1.0 - x)
        vals.append(x)
    return (np.array(vals) > 0.5).astype(np.float32).reshape(bs, bs)


def _dct2_kernel(a_ref, p_ref, pt_ref, o_ref):
    A = a_ref[0]
    t = jnp.dot(p_ref[...], A, preferred_element_type=jnp.float32)
    o_ref[0] = jnp.dot(t, pt_ref[...], preferred_element_type=jnp.float32)


def _dct2_pallas(imgs, P, PT):
    N, H, W = imgs.shape
    return pl.pallas_call(
        _dct2_kernel,
        grid=(N,),
        in_specs=[
            pl.BlockSpec((1, H, W), lambda i: (i, 0, 0)),
            pl.BlockSpec((H, H), lambda i: (0, 0)),
            pl.BlockSpec((H, H), lambda i: (0, 0)),
        ],
        out_specs=pl.BlockSpec((1, H, W), lambda i: (i, 0, 0)),
        out_shape=jax.ShapeDtypeStruct((N, H, W), jnp.float32),
    )(imgs, P, PT)


def kernel(cover, secret_bits):
    B, C, H, W = cover.shape
    bs = _BLOCK
    Dnp = _dct_matrix_np(bs)
    nb = H // bs
    P = jnp.asarray(np.kron(np.eye(nb, dtype=np.float32), Dnp))
    PT = jnp.asarray(np.kron(np.eye(nb, dtype=np.float32), Dnp.T.copy()))

    imgs = cover.reshape(B * C, H, W)
    dct_img = _dct2_pallas(imgs, P, PT)
    dct_blocks = (
        dct_img.reshape(B, C, nb, bs, nb, bs).transpose(0, 1, 2, 4, 3, 5)
    )

    D = jnp.asarray(Dnp)
    a = jnp.abs(dct_blocks)
    bmax = jnp.max(a, axis=(-2, -1), keepdims=True)
    e = a / (bmax + 1e-8)
    mask = ((e >= _MIN_E) & (e <= _MAX_E)).astype(jnp.float32)
    mask = mask.at[..., 0, 0].set(0.0)
    mask = mask * jnp.asarray(_chaotic_mask_np(bs))
    tv = jnp.var(_blockify(cover, bs), axis=(-2, -1))
    vn = (tv - tv.min()) / (tv.max() - tv.min() + 1e-8)
    thr = jnp.quantile(vn.reshape(-1), 0.3)
    tmask = (vn > thr).astype(jnp.float32)[..., None, None]
    mask = mask * tmask
    num_bits = secret_bits.shape[1]
    flat_mask = mask.reshape(-1) > 0
    total = flat_mask.shape[0]
    per_batch = total // B
    m_i32 = flat_mask.reshape(-1, bs * bs).astype(jnp.int32)
    local = jnp.cumsum(m_i32, axis=1)
    block_counts = local[:, -1]
    block_offsets = jnp.cumsum(block_counts) - block_counts
    rank = (block_offsets[:, None] + local).reshape(-1) - 1
    selected = flat_mask & (rank < num_bits)
    b_idx_all = jnp.arange(total) // per_batch
    rank_safe = jnp.clip(rank, 0, num_bits - 1)
    bits_all = secret_bits[b_idx_all, rank_safe].astype(jnp.float32)
    flat = dct_blocks.reshape(-1)
    c = flat
    rounded = jnp.round(c)
    lsb = jnp.mod(jnp.abs(rounded), 2.0)
    need = jnp.not_equal(lsb, bits_all)
    delta = jnp.where(selected & need, _STRENGTH * (2.0 * bits_all - 1.0) * jnp.where(c >= 0, 1.0, -1.0), 0.0)
    flat = flat + delta
    modified_blocks = flat.reshape(mask.shape)
    emap = selected.astype(jnp.float32).reshape(mask.shape)
    modified_dct = _unblockify(modified_blocks, H, W)
    mimgs = modified_dct.reshape(B * C, H, W)
    stego_img = _dct2_pallas(mimgs, PT, P)
    stego = stego_img.reshape(B, C, H, W)
    return stego, emap
